# baseline (speedup 1.0000x reference)
import functools

import jax
import jax.numpy as jnp
from jax.experimental import pallas as pl
from jax.experimental.pallas import tpu as pltpu

_KT = 512
_NB = 8


def _vq_body(last_ref, x_ref, e_ref, res_ref, idx_ref, *, M, K, C):
    E = e_ref[...]
    e2 = jnp.sum(E * E, axis=1)[None, :]
    lastv = last_ref[...]

    for i in range(_NB):
        X = x_ref[i]
        hx2 = 0.5 * jnp.sum(X * X, axis=1, keepdims=True)

        tiles = []
        for kt in range(K // _KT):
            E_t = e_ref[kt * _KT:(kt + 1) * _KT, :]
            p_t = jax.lax.dot_general(X, E_t, (((1,), (1,)), ((), ())),
                                      preferred_element_type=jnp.float32)
            tiles.append(jnp.min(hx2 - p_t, axis=0, keepdims=True))
        colmin = 2.0 * jnp.concatenate(tiles, axis=1) + e2

        gmin = jnp.min(colmin)
        kiota = jax.lax.broadcasted_iota(jnp.int32, (1, K), 1)
        k_star = jnp.min(jnp.where(colmin == gmin, kiota, K))

        e_row = e_ref[pl.ds(k_star, 1), :]
        gcol = hx2 - jnp.sum(X * e_row, axis=1, keepdims=True)
        cmin = jnp.min(gcol)
        miota = jax.lax.broadcasted_iota(jnp.int32, (M, 1), 0)
        m_star = jnp.min(jnp.where(gcol == cmin, miota, M))

        res_ref[i] = (x_ref[i, pl.ds(m_star, 1), :] - e_row) * lastv
        idx_ref[pl.ds(i, 1), :] = jnp.reshape(k_star, (1, 1))


def kernel(x, argmin, last, embed):
    del argmin
    N, M, C = x.shape
    K = embed.shape[0]
    lastv = jnp.asarray(last, x.dtype).reshape(1, 1)
    body = functools.partial(_vq_body, M=M, K=K, C=C)
    res, idx = pl.pallas_call(
        body,
        grid=(N // _NB,),
        in_specs=[
            pl.BlockSpec((1, 1), lambda n: (0, 0)),
            pl.BlockSpec((_NB, M, C), lambda n: (n, 0, 0)),
            pl.BlockSpec((K, C), lambda n: (0, 0)),
        ],
        out_specs=[
            pl.BlockSpec((_NB, 1, C), lambda n: (n, 0, 0)),
            pl.BlockSpec((_NB, 1), lambda n: (n, 0)),
        ],
        out_shape=[
            jax.ShapeDtypeStruct((N, 1, C), x.dtype),
            jax.ShapeDtypeStruct((N, 1), jnp.int32),
        ],
        compiler_params=pltpu.CompilerParams(
            dimension_semantics=("parallel",)),
    )(lastv, x, embed)
    return res, idx


# speedup vs baseline: 1.0100x; 1.0100x over previous
import functools

import jax
import jax.numpy as jnp
from jax.experimental import pallas as pl
from jax.experimental.pallas import tpu as pltpu

_KT = 256
_NB = 8


def _vq_body(last_ref, x_ref, e_ref, res_ref, idx_ref, *, M, K, C):
    E = e_ref[...]
    e2 = jnp.sum(E * E, axis=1)[None, :]
    lastv = last_ref[...]

    for i in range(_NB):
        X = x_ref[i]
        hx2 = 0.5 * jnp.sum(X * X, axis=1, keepdims=True)

        tiles = []
        for kt in range(K // _KT):
            E_t = e_ref[kt * _KT:(kt + 1) * _KT, :]
            p_t = jax.lax.dot_general(X, E_t, (((1,), (1,)), ((), ())),
                                      preferred_element_type=jnp.float32)
            tiles.append(jnp.min(hx2 - p_t, axis=0, keepdims=True))
        colmin = 2.0 * jnp.concatenate(tiles, axis=1) + e2

        gmin = jnp.min(colmin)
        kiota = jax.lax.broadcasted_iota(jnp.int32, (1, K), 1)
        k_star = jnp.min(jnp.where(colmin == gmin, kiota, K))

        e_row = e_ref[pl.ds(k_star, 1), :]
        gcol = hx2 - jnp.sum(X * e_row, axis=1, keepdims=True)
        cmin = jnp.min(gcol)
        miota = jax.lax.broadcasted_iota(jnp.int32, (M, 1), 0)
        m_star = jnp.min(jnp.where(gcol == cmin, miota, M))

        res_ref[i] = (x_ref[i, pl.ds(m_star, 1), :] - e_row) * lastv
        idx_ref[pl.ds(i, 1), :] = jnp.reshape(k_star, (1, 1))


def kernel(x, argmin, last, embed):
    del argmin
    N, M, C = x.shape
    K = embed.shape[0]
    lastv = jnp.asarray(last, x.dtype).reshape(1, 1)
    body = functools.partial(_vq_body, M=M, K=K, C=C)
    res, idx = pl.pallas_call(
        body,
        grid=(N // _NB,),
        in_specs=[
            pl.BlockSpec((1, 1), lambda n: (0, 0)),
            pl.BlockSpec((_NB, M, C), lambda n: (n, 0, 0)),
            pl.BlockSpec((K, C), lambda n: (0, 0)),
        ],
        out_specs=[
            pl.BlockSpec((_NB, 1, C), lambda n: (n, 0, 0)),
            pl.BlockSpec((_NB, 1), lambda n: (n, 0)),
        ],
        out_shape=[
            jax.ShapeDtypeStruct((N, 1, C), x.dtype),
            jax.ShapeDtypeStruct((N, 1), jnp.int32),
        ],
        compiler_params=pltpu.CompilerParams(
            dimension_semantics=("parallel",),
            allow_input_fusion=[True, False, False]),
    )(lastv, x, embed)
    return res, idx
